# baseline (speedup 1.0000x reference)
import functools

import jax
import jax.numpy as jnp
from jax import lax
from jax.experimental import pallas as pl
from jax.experimental.pallas import tpu as pltpu
from jax.experimental.pallas import tpu_sc as plsc

QT = 1024
NQB = 1
Q = QT // NQB
D = 16
N = 100000
K = 5
CHUNK = 8192
NPAD = 106496
NCHUNKS = NPAD // CHUNK
BIG = 3.0e38
BIGF = 3.0e38


def _topk_body(q_ref, s_ref, idx_out_ref, w_out_ref, runv_ref, runi_ref):
    i = pl.program_id(1)

    q = q_ref[...]
    qn = jnp.sqrt(jnp.sum(q * q, axis=1, keepdims=True))
    qh = q / jnp.maximum(qn, 1e-12)
    q_sq = jnp.sum(qh * qh, axis=1, keepdims=True)

    s = s_ref[...]
    sn = jnp.sqrt(jnp.sum(s * s, axis=1, keepdims=True))
    sh = s / jnp.maximum(sn, 1e-12)
    s_sq = jnp.sum(sh * sh, axis=1, keepdims=True)

    qa = jnp.concatenate([qh * -2.0, jnp.ones((Q, 1), jnp.float32)], axis=1)
    sa = jnp.concatenate([sh, s_sq], axis=1)
    sq = lax.dot_general(qa, sa, (((1,), (1,)), ((), ())),
                         preferred_element_type=jnp.float32)

    lanes = lax.broadcasted_iota(jnp.int32, (Q, CHUNK), 1).astype(jnp.float32)
    base_f = (i * CHUNK).astype(jnp.float32)
    sq = jnp.where(lanes + base_f >= N, BIG, sq)

    vals, idxs = [], []
    cur = sq
    for t in range(K):
        m = jnp.min(cur, axis=1, keepdims=True)
        sel = jnp.min(jnp.where(cur == m, lanes, BIGF), axis=1, keepdims=True)
        vals.append(m)
        idxs.append(sel + base_f)
        if t < K - 1:
            cur = jnp.where(lanes == sel, BIG, cur)
    padv = jnp.full((Q, 1), BIG, jnp.float32)
    padi = jnp.zeros((Q, 1), jnp.float32)
    cv = jnp.concatenate(vals + [padv, padv, padv], axis=1)
    ci = jnp.concatenate(idxs + [padi, padi, padi], axis=1)

    @pl.when(i == 0)
    def _init():
        runv_ref[...] = jnp.full((Q, 8), BIG, jnp.float32)
        runi_ref[...] = jnp.zeros((Q, 8), jnp.float32)

    mv = jnp.concatenate([runv_ref[...], cv], axis=1)
    mi = jnp.concatenate([runi_ref[...], ci], axis=1)
    slot = lax.broadcasted_iota(jnp.int32, (Q, 16), 1).astype(jnp.float32)
    nv, ni = [], []
    curm = mv
    for _ in range(K):
        m = jnp.min(curm, axis=1, keepdims=True)
        ssel = jnp.min(jnp.where(curm == m, slot, BIGF), axis=1, keepdims=True)
        gi = jnp.max(jnp.where(slot == ssel, mi, -1.0), axis=1, keepdims=True)
        nv.append(m)
        ni.append(gi)
        curm = jnp.where(slot == ssel, BIG, curm)
    runv_ref[...] = jnp.concatenate(nv + [padv, padv, padv], axis=1)
    runi_ref[...] = jnp.concatenate(ni + [padi, padi, padi], axis=1)

    @pl.when(i == NCHUNKS - 1)
    def _fin():
        v = runv_ref[...] + q_sq
        d = jnp.sqrt(jnp.maximum(v, 1e-12))
        w = 1.0 / (d + 1e-8)
        kmask = lax.broadcasted_iota(jnp.int32, (Q, 8), 1) < K
        w = jnp.where(kmask, w, 0.0)
        w_out_ref[...] = w / jnp.sum(w, axis=1, keepdims=True)
        idx_out_ref[...] = jnp.where(
            kmask, runi_ref[...].astype(jnp.int32), 0)


def _tc_topk(query, supports_padded, interpret=False):
    return pl.pallas_call(
        _topk_body,
        grid=(NQB, NCHUNKS),
        in_specs=[
            pl.BlockSpec((Q, D), lambda j, i: (j, 0)),
            pl.BlockSpec((CHUNK, D), lambda j, i: (i, 0)),
        ],
        out_specs=[
            pl.BlockSpec((Q, 8), lambda j, i: (j, 0)),
            pl.BlockSpec((Q, 8), lambda j, i: (j, 0)),
        ],
        out_shape=[
            jax.ShapeDtypeStruct((QT, 8), jnp.int32),
            jax.ShapeDtypeStruct((QT, 8), jnp.float32),
        ],
        scratch_shapes=[
            pltpu.VMEM((Q, 8), jnp.float32),
            pltpu.VMEM((Q, 8), jnp.float32),
        ],
        compiler_params=pltpu.CompilerParams(
            dimension_semantics=("parallel", "arbitrary")),
        interpret=interpret,
    )(query, supports_padded)


try:
    _SC_INFO = plsc.get_sparse_core_info()
    _NC = _SC_INFO.num_cores
    _NS = _SC_INFO.num_subcores
except Exception:
    _NC, _NS = 2, 16
_NW = _NC * _NS
_ROWS = QT * 8
_RW = _ROWS // _NW
_QW = QT // _NW


def _sc_combine_body(tgt_hbm, idx_hbm, wexp_hbm, out_hbm,
                     idx_v, rows_v, w_v, acc_v, sem):
    wid = lax.axis_index("s") * _NC + lax.axis_index("c")
    nslab = _RW // 128
    pltpu.sync_copy(idx_hbm.at[pl.ds(wid * nslab, nslab)], idx_v)
    for j in range(nslab):
        pltpu.async_copy(tgt_hbm.at[idx_v.at[j]],
                         rows_v.at[pl.ds(j * 128, 128)], sem).wait()
    pltpu.sync_copy(wexp_hbm.at[pl.ds(wid * _RW, _RW)], w_v)
    for qq in range(_QW):
        acc = rows_v[qq * 8, :] * w_v[qq * 8, :]
        for kk in range(1, 8):
            acc = acc + rows_v[qq * 8 + kk, :] * w_v[qq * 8 + kk, :]
        acc_v[qq, :] = acc
    pltpu.sync_copy(acc_v, out_hbm.at[pl.ds(wid * _QW, _QW)])


@functools.lru_cache(maxsize=1)
def _sc_combine_kernel():
    return pl.kernel(
        _sc_combine_body,
        mesh=plsc.VectorSubcoreMesh(core_axis_name="c", subcore_axis_name="s"),
        out_type=jax.ShapeDtypeStruct((QT, D), jnp.float32),
        scratch_types=[
            pltpu.VMEM((_RW // 128, 128), jnp.int32),
            pltpu.VMEM((_RW, D), jnp.float32),
            pltpu.VMEM((_RW, D), jnp.float32),
            pltpu.VMEM((_QW, D), jnp.float32),
            pltpu.SemaphoreType.DMA,
        ],
        compiler_params=pltpu.CompilerParams(use_tc_tiling_on_sc=False),
    )


def kernel(query, supports, targets, k):
    del k
    supports_padded = jnp.pad(supports, ((0, NPAD - N), (0, 0)))
    idx8, w8 = _tc_topk(query, supports_padded)
    idx_slab = idx8.reshape(_ROWS // 128, 128)
    w_exp = jnp.broadcast_to(w8.reshape(_ROWS, 1), (_ROWS, D))
    return _sc_combine_kernel()(targets, idx_slab, w_exp)


# speedup vs baseline: 1.2096x; 1.2096x over previous
import functools

import jax
import jax.numpy as jnp
from jax import lax
from jax.experimental import pallas as pl
from jax.experimental.pallas import tpu as pltpu
from jax.experimental.pallas import tpu_sc as plsc

QT = 1024
NQB = 1
Q = QT // NQB
D = 16
N = 100000
K = 5
CHUNK = 4096
NPAD = 102400
NCHUNKS = NPAD // CHUNK
BIG = 3.0e38
BIGF = 3.0e38


def _topk_body(q_ref, s_ref, idx_out_ref, w_out_ref, runv_ref, runi_ref):
    i = pl.program_id(1)

    q = q_ref[...]
    qn = jnp.sqrt(jnp.sum(q * q, axis=1, keepdims=True))
    qh = q / jnp.maximum(qn, 1e-12)
    q_sq = jnp.sum(qh * qh, axis=1, keepdims=True)

    s = s_ref[...]
    sn = jnp.sqrt(jnp.sum(s * s, axis=1, keepdims=True))
    sh = s / jnp.maximum(sn, 1e-12)
    s_sq = jnp.sum(sh * sh, axis=1, keepdims=True)

    qa = jnp.concatenate([qh * -2.0, jnp.ones((Q, 1), jnp.float32)], axis=1)
    sa = jnp.concatenate([sh, s_sq], axis=1)
    sq = lax.dot_general(qa, sa, (((1,), (1,)), ((), ())),
                         preferred_element_type=jnp.float32)

    lanes = lax.broadcasted_iota(jnp.int32, (Q, CHUNK), 1).astype(jnp.float32)
    base_f = (i * CHUNK).astype(jnp.float32)
    sq = jnp.where(lanes + base_f >= N, BIG, sq)

    vals, idxs = [], []
    cur = sq
    for t in range(K):
        m = jnp.min(cur, axis=1, keepdims=True)
        sel = jnp.min(jnp.where(cur == m, lanes, BIGF), axis=1, keepdims=True)
        vals.append(m)
        idxs.append(sel + base_f)
        if t < K - 1:
            cur = jnp.where(lanes == sel, BIG, cur)
    padv = jnp.full((Q, 1), BIG, jnp.float32)
    padi = jnp.zeros((Q, 1), jnp.float32)
    cv = jnp.concatenate(vals + [padv, padv, padv], axis=1)
    ci = jnp.concatenate(idxs + [padi, padi, padi], axis=1)

    @pl.when(i == 0)
    def _init():
        runv_ref[...] = jnp.full((Q, 8), BIG, jnp.float32)
        runi_ref[...] = jnp.zeros((Q, 8), jnp.float32)

    mv = jnp.concatenate([runv_ref[...], cv], axis=1)
    mi = jnp.concatenate([runi_ref[...], ci], axis=1)
    slot = lax.broadcasted_iota(jnp.int32, (Q, 16), 1).astype(jnp.float32)
    nv, ni = [], []
    curm = mv
    for _ in range(K):
        m = jnp.min(curm, axis=1, keepdims=True)
        ssel = jnp.min(jnp.where(curm == m, slot, BIGF), axis=1, keepdims=True)
        gi = jnp.max(jnp.where(slot == ssel, mi, -1.0), axis=1, keepdims=True)
        nv.append(m)
        ni.append(gi)
        curm = jnp.where(slot == ssel, BIG, curm)
    runv_ref[...] = jnp.concatenate(nv + [padv, padv, padv], axis=1)
    runi_ref[...] = jnp.concatenate(ni + [padi, padi, padi], axis=1)

    @pl.when(i == NCHUNKS - 1)
    def _fin():
        v = runv_ref[...] + q_sq
        d = jnp.sqrt(jnp.maximum(v, 1e-12))
        w = 1.0 / (d + 1e-8)
        kmask = lax.broadcasted_iota(jnp.int32, (Q, 8), 1) < K
        w = jnp.where(kmask, w, 0.0)
        w_out_ref[...] = w / jnp.sum(w, axis=1, keepdims=True)
        idx_out_ref[...] = jnp.where(
            kmask, runi_ref[...].astype(jnp.int32), 0)


def _tc_topk(query, supports_padded, interpret=False):
    return pl.pallas_call(
        _topk_body,
        grid=(NQB, NCHUNKS),
        in_specs=[
            pl.BlockSpec((Q, D), lambda j, i: (j, 0)),
            pl.BlockSpec((CHUNK, D), lambda j, i: (i, 0)),
        ],
        out_specs=[
            pl.BlockSpec((Q, 8), lambda j, i: (j, 0)),
            pl.BlockSpec((Q, 8), lambda j, i: (j, 0)),
        ],
        out_shape=[
            jax.ShapeDtypeStruct((QT, 8), jnp.int32),
            jax.ShapeDtypeStruct((QT, 8), jnp.float32),
        ],
        scratch_shapes=[
            pltpu.VMEM((Q, 8), jnp.float32),
            pltpu.VMEM((Q, 8), jnp.float32),
        ],
        compiler_params=pltpu.CompilerParams(
            dimension_semantics=("parallel", "arbitrary")),
        interpret=interpret,
    )(query, supports_padded)


try:
    _SC_INFO = plsc.get_sparse_core_info()
    _NC = _SC_INFO.num_cores
    _NS = _SC_INFO.num_subcores
except Exception:
    _NC, _NS = 2, 16
_NW = _NC * _NS
_ROWS = QT * 8
_RW = _ROWS // _NW
_QW = QT // _NW


def _sc_combine_body(tgt_hbm, idx_hbm, wexp_hbm, out_hbm,
                     idx_v, rows_v, w_v, acc_v, sem):
    wid = lax.axis_index("s") * _NC + lax.axis_index("c")
    nslab = _RW // 128
    pltpu.sync_copy(idx_hbm.at[pl.ds(wid * nslab, nslab)], idx_v)
    for j in range(nslab):
        pltpu.async_copy(tgt_hbm.at[idx_v.at[j]],
                         rows_v.at[pl.ds(j * 128, 128)], sem).wait()
    pltpu.sync_copy(wexp_hbm.at[pl.ds(wid * _RW, _RW)], w_v)
    for qq in range(_QW):
        acc = rows_v[qq * 8, :] * w_v[qq * 8, :]
        for kk in range(1, 8):
            acc = acc + rows_v[qq * 8 + kk, :] * w_v[qq * 8 + kk, :]
        acc_v[qq, :] = acc
    pltpu.sync_copy(acc_v, out_hbm.at[pl.ds(wid * _QW, _QW)])


@functools.lru_cache(maxsize=1)
def _sc_combine_kernel():
    return pl.kernel(
        _sc_combine_body,
        mesh=plsc.VectorSubcoreMesh(core_axis_name="c", subcore_axis_name="s"),
        out_type=jax.ShapeDtypeStruct((QT, D), jnp.float32),
        scratch_types=[
            pltpu.VMEM((_RW // 128, 128), jnp.int32),
            pltpu.VMEM((_RW, D), jnp.float32),
            pltpu.VMEM((_RW, D), jnp.float32),
            pltpu.VMEM((_QW, D), jnp.float32),
            pltpu.SemaphoreType.DMA,
        ],
        compiler_params=pltpu.CompilerParams(use_tc_tiling_on_sc=False),
    )


def kernel(query, supports, targets, k):
    del k
    supports_padded = jnp.pad(supports, ((0, NPAD - N), (0, 0)))
    idx8, w8 = _tc_topk(query, supports_padded)
    idx_slab = idx8.reshape(_ROWS // 128, 128)
    w_exp = jnp.broadcast_to(w8.reshape(_ROWS, 1), (_ROWS, D))
    return _sc_combine_kernel()(targets, idx_slab, w_exp)
